# Initial kernel scaffold; baseline (speedup 1.0000x reference)
import jax
import jax.numpy as jnp
from jax.experimental import pallas as pl


def kernel(image_features, edge_index, non_image_features, edge_index1, edge_weight1, params):
    raise NotImplementedError("write your pallas kernel here")


# speedup vs baseline: 5.9250x; 5.9250x over previous
import functools

import jax
import jax.numpy as jnp
import numpy as np
from jax import lax
from jax.experimental import pallas as pl
from jax.experimental.pallas import tpu as pltpu
from jax.experimental.pallas import tpu_sc as plsc

N = 10000
E = 160000
IN_DIM = 512
EMB = 256
HID = 16

NC, NS, L = 2, 16, 16
NW = NC * NS
E_PAD = 163840
EPW = E_PAD // NW
C = 64
NCH = EPW // C
NPT = N // NS

_BN_EPS = 1e-5

_CP = pltpu.CompilerParams(use_tc_tiling_on_sc=False, needs_layout_passes=False)


@functools.cache
def _mesh():
    return plsc.VectorSubcoreMesh(core_axis_name="c", subcore_axis_name="s")

f32 = jnp.float32
i32 = jnp.int32
SDS = jax.ShapeDtypeStruct


def _wid():
    return lax.axis_index("s") * NC + lax.axis_index("c")


def _tc1_body(x_ref, w0_ref, b0_ref, w1_ref, b1_ref, wc_ref,
              ei_ref, nsq_ref, y0_ref, t1_ref):
    x = x_ref[...]
    h = jnp.dot(x, w0_ref[...], preferred_element_type=f32) + b0_ref[...]
    ei = jnp.dot(h, w1_ref[...], preferred_element_type=f32) + b1_ref[...]
    ei_ref[...] = ei
    nsq_ref[...] = jnp.sum(ei * ei, axis=1, keepdims=True)
    wc = wc_ref[...]
    y = jnp.dot(ei, wc, preferred_element_type=f32)
    y0_ref[...] = y[:, 0:16]
    t1_ref[0] = y[:, 16:32]
    t1_ref[1] = y[:, 32:48]


def _tc1(x, w0, b0, w1, b1, wcat):
    R = 1000
    return pl.pallas_call(
        _tc1_body,
        grid=(N // R,),
        in_specs=[
            pl.BlockSpec((R, IN_DIM), lambda i: (i, 0)),
            pl.BlockSpec((IN_DIM, IN_DIM), lambda i: (0, 0)),
            pl.BlockSpec((1, IN_DIM), lambda i: (0, 0)),
            pl.BlockSpec((IN_DIM, EMB), lambda i: (0, 0)),
            pl.BlockSpec((1, EMB), lambda i: (0, 0)),
            pl.BlockSpec((EMB, 48), lambda i: (0, 0)),
        ],
        out_specs=[
            pl.BlockSpec((R, EMB), lambda i: (i, 0)),
            pl.BlockSpec((R, 1), lambda i: (i, 0)),
            pl.BlockSpec((R, 16), lambda i: (i, 0)),
            pl.BlockSpec((2, R, 16), lambda i: (0, i, 0)),
        ],
        out_shape=[
            SDS((N, EMB), f32),
            SDS((N, 1), f32),
            SDS((N, 16), f32),
            SDS((2, N, 16), f32),
        ],
    )(x, w0, b0[None], w1, b1[None], wcat)


@functools.cache
def _sc2_kernel():
  deco = functools.partial(
    pl.kernel, mesh=_mesh(), compiler_params=_CP,
    out_type=(SDS((E_PAD,), f32), SDS((E_PAD,), f32), SDS((E_PAD,), f32)),
    scratch_types=[
        pltpu.VMEM((N,), f32),
        pltpu.VMEM((EPW,), i32),
        pltpu.VMEM((EPW,), i32),
        pltpu.VMEM((EPW,), f32),
        pltpu.VMEM((EPW,), f32),
        pltpu.VMEM((EPW,), f32),
        pltpu.VMEM((2, C, EMB), f32),
        pltpu.VMEM((2, C, EMB), f32),
        pltpu.VMEM((C, 16), f32),
        pltpu.SemaphoreType.DMA,
        pltpu.SemaphoreType.DMA,
        pltpu.SemaphoreType.DMA,
        pltpu.SemaphoreType.DMA,
    ],
)
  def _sc2(es_hbm, nsq_hbm, src_hbm, dst_hbm,
           dot_o, nsqs_o, nsqd_o,
           nsqb, srcb, dstb, dots, nsqsb, nsqdb, rs, rd, dtmp,
           gs0, gs1, gd0, gd1):
      base = _wid() * EPW
      pltpu.sync_copy(nsq_hbm, nsqb)
      pltpu.sync_copy(src_hbm.at[pl.ds(base, EPW)], srcb)
      pltpu.sync_copy(dst_hbm.at[pl.ds(base, EPW)], dstb)

      @pl.loop(0, EPW // 16)
      def _(g):
          s16 = srcb[pl.ds(g * 16, 16)]
          d16 = dstb[pl.ds(g * 16, 16)]
          nsqsb[pl.ds(g * 16, 16)] = plsc.load_gather(nsqb, [s16])
          nsqdb[pl.ds(g * 16, 16)] = plsc.load_gather(nsqb, [d16])

      gsems = (gs0, gs1)
      gdems = (gd0, gd1)
      for b in range(2):
          pltpu.async_copy(es_hbm.at[srcb.at[pl.ds(b * C, C)]], rs.at[b], gsems[b])
          pltpu.async_copy(es_hbm.at[dstb.at[pl.ds(b * C, C)]], rd.at[b], gdems[b])

      @pl.loop(0, NCH // 2)
      def _(k2):
          for b in range(2):
              k = k2 * 2 + b
              pltpu.make_async_copy(es_hbm.at[srcb.at[pl.ds(k * C, C)]],
                                    rs.at[b], gsems[b]).wait()
              pltpu.make_async_copy(es_hbm.at[dstb.at[pl.ds(k * C, C)]],
                                    rd.at[b], gdems[b]).wait()
              rs_b = rs.at[b]
              rd_b = rd.at[b]

              @pl.loop(0, C)
              def _(e):
                  acc = rs_b[e, pl.ds(0, 16)] * rd_b[e, pl.ds(0, 16)]
                  for q in range(1, EMB // 16):
                      acc = acc + (rs_b[e, pl.ds(q * 16, 16)]
                                   * rd_b[e, pl.ds(q * 16, 16)])
                  dtmp[e] = plsc.cumsum(acc)

              lane15 = jnp.full((16,), 15, i32)
              ramp = lax.iota(i32, 16)
              for g in range(C // 16):
                  dots[pl.ds(k * C + g * 16, 16)] = plsc.load_gather(
                      dtmp, [ramp + g * 16, lane15])

              @pl.when(k + 2 < NCH)
              def _():
                  pltpu.async_copy(es_hbm.at[srcb.at[pl.ds((k + 2) * C, C)]],
                                   rs.at[b], gsems[b])
                  pltpu.async_copy(es_hbm.at[dstb.at[pl.ds((k + 2) * C, C)]],
                                   rd.at[b], gdems[b])

      pltpu.sync_copy(dots, dot_o.at[pl.ds(base, EPW)])
      pltpu.sync_copy(nsqsb, nsqs_o.at[pl.ds(base, EPW)])
      pltpu.sync_copy(nsqdb, nsqd_o.at[pl.ds(base, EPW)])
  return deco(_sc2)


def _tc3_body(nim_ref, dot_ref, nsqs_ref, nsqd_ref,
              w0_ref, b0_ref, gh_ref, bt_ref, w1_ref, b1_ref,
              ew_ref):
    i = pl.program_id(0)
    x = nim_ref[...]
    w0 = w0_ref[...]
    b0 = b0_ref[...]
    gh = gh_ref[...]
    bt = bt_ref[...]
    w1 = w1_ref[...]
    b1 = b1_ref[...]

    def parser(a, b):
        h = jnp.maximum(a * w0[0:1, :] + b * w0[1:2, :] + b0, 0.0)
        h = h * gh + bt
        return jnp.dot(h, w1, preferred_element_type=f32) + b1

    p1 = parser(x[:, 0:1], x[:, 1:2])
    p2 = parser(x[:, 2:3], x[:, 3:4])
    s12 = jnp.sum(p1 * p2, axis=1, keepdims=True)
    s11 = jnp.sum(p1 * p1, axis=1, keepdims=True)
    s22 = jnp.sum(p2 * p2, axis=1, keepdims=True)
    num = s12 + dot_ref[...]
    n1 = jnp.maximum(jnp.sqrt(s11 + nsqs_ref[...]), 1e-8)
    n2 = jnp.maximum(jnp.sqrt(s22 + nsqd_ref[...]), 1e-8)
    ew = (num / (n1 * n2) + 1.0) * 0.5
    B = ew.shape[0]
    ridx = i * B + lax.broadcasted_iota(i32, (B, 1), 0)
    ew_ref[...] = jnp.where(ridx < E, ew, 0.0)


def _tc3(nim, dote, nsqs, nsqd, p):
    B = 2048
    gh = (p['pg'] / jnp.sqrt(1.0 + _BN_EPS))[None]
    col = lambda i: (i, 0)
    return pl.pallas_call(
        _tc3_body,
        grid=(E_PAD // B,),
        in_specs=[
            pl.BlockSpec((B, 4), col),
            pl.BlockSpec((B, 1), col),
            pl.BlockSpec((B, 1), col),
            pl.BlockSpec((B, 1), col),
            pl.BlockSpec((2, 128), lambda i: (0, 0)),
            pl.BlockSpec((1, 128), lambda i: (0, 0)),
            pl.BlockSpec((1, 128), lambda i: (0, 0)),
            pl.BlockSpec((1, 128), lambda i: (0, 0)),
            pl.BlockSpec((128, 128), lambda i: (0, 0)),
            pl.BlockSpec((1, 128), lambda i: (0, 0)),
        ],
        out_specs=pl.BlockSpec((B, 1), col),
        out_shape=SDS((E_PAD, 1), f32),
    )(nim, dote, nsqs, nsqd, p['pW0'], p['pb0'][None], gh, p['pbt'][None],
      p['pW1'], p['pb1'][None])


@functools.cache
def _sc4_kernel():
  deco = functools.partial(
    pl.kernel, mesh=_mesh(), compiler_params=_CP,
    out_type=SDS((NW, N), f32),
    scratch_types=[
        pltpu.VMEM((N,), f32),
        pltpu.VMEM((EPW,), i32),
        pltpu.VMEM((EPW,), f32),
    ],
)
  def _sc4(src_hbm, ew_hbm, degp_o, degb, srcb, ewb):
      w = _wid()
      base = w * EPW
      pltpu.sync_copy(src_hbm.at[pl.ds(base, EPW)], srcb)
      pltpu.sync_copy(ew_hbm.at[pl.ds(base, EPW)], ewb)

      @pl.loop(0, N // 16)
      def _(r):
          degb[pl.ds(r * 16, 16)] = jnp.zeros((16,), f32)

      @pl.loop(0, EPW // 16)
      def _(g):
          idx = srcb[pl.ds(g * 16, 16)]
          val = ewb[pl.ds(g * 16, 16)]
          plsc.addupdate_scatter(degb, [idx], val)

      pltpu.sync_copy(degb, degp_o.at[w])
  return deco(_sc4)


def _tc5_body(degp_ref, dis_ref):
    deg = jnp.sum(degp_ref[...], axis=0, keepdims=True)
    safe = jnp.where(deg > 0, deg, 1.0)
    dis_ref[...] = jnp.where(deg > 0, lax.rsqrt(safe), 0.0)


def _tc5(degp):
    return pl.pallas_call(
        _tc5_body,
        out_shape=SDS((1, N), f32),
    )(degp)


@functools.cache
def _sc6_kernel():
  deco = functools.partial(
    pl.kernel, mesh=_mesh(), compiler_params=_CP,
    out_type=SDS((E_PAD,), f32),
    scratch_types=[
        pltpu.VMEM((N,), f32),
        pltpu.VMEM((EPW,), i32),
        pltpu.VMEM((EPW,), i32),
        pltpu.VMEM((EPW,), f32),
        pltpu.VMEM((EPW,), f32),
    ],
)
  def _sc6(dis_hbm, src_hbm, dst_hbm, ew_hbm, norm_o,
           disb, srcb, dstb, ewb, normb):
      base = _wid() * EPW
      pltpu.sync_copy(dis_hbm, disb)
      pltpu.sync_copy(src_hbm.at[pl.ds(base, EPW)], srcb)
      pltpu.sync_copy(dst_hbm.at[pl.ds(base, EPW)], dstb)
      pltpu.sync_copy(ew_hbm.at[pl.ds(base, EPW)], ewb)

      @pl.loop(0, EPW // 16)
      def _(g):
          s16 = srcb[pl.ds(g * 16, 16)]
          d16 = dstb[pl.ds(g * 16, 16)]
          a = plsc.load_gather(disb, [s16])
          b = plsc.load_gather(disb, [d16])
          normb[pl.ds(g * 16, 16)] = -a * ewb[pl.ds(g * 16, 16)] * b

      pltpu.sync_copy(normb, norm_o.at[pl.ds(base, EPW)])
  return deco(_sc6)


def _make_prop(combine):
    n_acc = 1 if combine else 2
    out_t = (SDS((NC, N, 16), f32) if combine
             else (SDS((NC, N, 16), f32), SDS((NC, N, 16), f32)))
    scr = [
        pltpu.VMEM((EPW,), i32),
        pltpu.VMEM((EPW,), i32),
        pltpu.VMEM((NCH, C), i32),
        pltpu.VMEM((EPW,), f32),
        pltpu.VMEM((NPT, 16), f32),
        pltpu.VMEM((2, C, 16), f32),
        pltpu.VMEM((2, C, 16), f32),
        pltpu.VMEM((2, C, 16), f32),
    ]
    if not combine:
        scr.append(pltpu.VMEM((2, C, 16), f32))
    scr += [pltpu.SemaphoreType.DMA] * (6 if combine else 8)
    scr.append(pltpu.VMEM_SHARED((N, 16), f32))
    if not combine:
        scr.append(pltpu.VMEM_SHARED((N, 16), f32))

    def body(table_hbm, src_hbm, dst_hbm, norm_hbm, *rest):
        if combine:
            (out1, srcb, srcb2, dstb, normb, zbuf, ga, gb, s1,
             sga0, sga1, sgb0, sgb1, ss0, ss1, acc1) = rest
            outs, accs, sbufs = (out1,), (acc1,), (s1,)
            ssems = ((ss0, ss1),)
        else:
            (out1, out2, srcb, srcb2, dstb, normb, zbuf, ga, gb, s1, s2,
             sga0, sga1, sgb0, sgb1, ss0, ss1, ss20, ss21,
             acc1, acc2) = rest
            outs, accs, sbufs = (out1, out2), (acc1, acc2), (s1, s2)
            ssems = ((ss0, ss1), (ss20, ss21))

        w = _wid()
        cid = lax.axis_index("c")
        sid = lax.axis_index("s")
        base = w * EPW

        pltpu.sync_copy(src_hbm.at[pl.ds(base, EPW)], srcb)
        pltpu.sync_copy(norm_hbm.at[pl.ds(base, EPW)], normb)
        pltpu.sync_copy(dst_hbm.at[w], dstb)

        @pl.loop(0, EPW // 16)
        def _(g):
            srcb2[pl.ds(g * 16, 16)] = srcb[pl.ds(g * 16, 16)] + N

        @pl.loop(0, NPT)
        def _(r):
            zbuf[r] = jnp.zeros((16,), f32)

        for acc in accs:
            pltpu.sync_copy(zbuf, acc.at[pl.ds(sid * NPT, NPT)])
        plsc.subcore_barrier()

        gsa = (sga0, sga1)
        gsb = (sgb0, sgb1)

        for b in range(2):
            pltpu.async_copy(table_hbm.at[srcb.at[pl.ds(b * C, C)]],
                             ga.at[b], gsa[b])
            pltpu.async_copy(table_hbm.at[srcb2.at[pl.ds(b * C, C)]],
                             gb.at[b], gsb[b])

        @pl.loop(0, NCH // 2)
        def _(k2):
            for b in range(2):
                k = k2 * 2 + b
                pltpu.make_async_copy(table_hbm.at[srcb.at[pl.ds(k * C, C)]],
                                      ga.at[b], gsa[b]).wait()
                pltpu.make_async_copy(table_hbm.at[srcb2.at[pl.ds(k * C, C)]],
                                      gb.at[b], gsb[b]).wait()

                @pl.when(k >= 2)
                def _():
                    for ai in range(n_acc):
                        pltpu.make_async_copy(
                            sbufs[ai].at[b],
                            accs[ai].at[dstb.at[k - 2]], ssems[ai][b]).wait()

                ga_b = ga.at[b]
                gb_b = gb.at[b]
                s1_b = sbufs[0].at[b]
                if not combine:
                    s2_b = sbufs[1].at[b]

                @pl.loop(0, C)
                def _(e):
                    nb = plsc.load_gather(
                        normb, [jnp.full((16,), 1, i32) * (k * C + e)])
                    if combine:
                        s1_b[e] = (ga_b[e] + gb_b[e]) * nb
                    else:
                        s1_b[e] = ga_b[e] * nb
                        s2_b[e] = gb_b[e] * nb

                @pl.when(k + 2 < NCH)
                def _():
                    pltpu.async_copy(
                        table_hbm.at[srcb.at[pl.ds((k + 2) * C, C)]],
                        ga.at[b], gsa[b])
                    pltpu.async_copy(
                        table_hbm.at[srcb2.at[pl.ds((k + 2) * C, C)]],
                        gb.at[b], gsb[b])

                for ai in range(n_acc):
                    pltpu.async_copy(sbufs[ai].at[b],
                                     accs[ai].at[dstb.at[k]], ssems[ai][b],
                                     add=True)

        for b in range(2):
            k = NCH - 2 + b
            for ai in range(n_acc):
                pltpu.make_async_copy(sbufs[ai].at[b],
                                      accs[ai].at[dstb.at[k]],
                                      ssems[ai][b]).wait()
        plsc.subcore_barrier()
        for ai in range(n_acc):
            pltpu.sync_copy(accs[ai].at[pl.ds(sid * NPT, NPT)],
                            outs[ai].at[cid].at[pl.ds(sid * NPT, NPT)])

    return functools.partial(pl.kernel, mesh=_mesh(), compiler_params=_CP,
                             out_type=out_t, scratch_types=scr)(body)


def _conv_flat(a, m0, mm, mp):
    up = jnp.concatenate([jnp.zeros((1, 128), f32), a[:-1]], axis=0)
    dn = jnp.concatenate([a[1:], jnp.zeros((1, 128), f32)], axis=0)
    return (jnp.dot(a, m0, preferred_element_type=f32)
            + jnp.dot(up, mm, preferred_element_type=f32)
            + jnp.dot(dn, mp, preferred_element_type=f32))


def _tc9_mid_body(y0_ref, y2_ref, p1_ref, q_ref, h1_ref,
                  m0_ref, mm_ref, mp_ref, cb_ref, bd0_ref, bd12_ref,
                  h1o_ref, y0o_ref, t1o_ref, first):
    cheb = (y0_ref[...] + p1_ref[0] + p1_ref[1]
            + 2.0 * (q_ref[0] + q_ref[1]) - y2_ref[...])
    a = jnp.maximum(cheb, 0.0)
    hh = _conv_flat(a, m0_ref[...], mm_ref[...], mp_ref[...]) + cb_ref[...]
    h1 = hh if first else h1_ref[...] + hh
    h1o_ref[...] = h1
    y0o_ref[...] = jnp.dot(h1, bd0_ref[...], preferred_element_type=f32)
    t = jnp.dot(h1, bd12_ref[...], preferred_element_type=f32)
    t1o_ref[0] = t[:, 0:128]
    t1o_ref[1] = t[:, 128:256]


def _tc9_mid(y0f, y2f, p1p, qp, h1f, m0, mm, mp, cb2, bd0, bd12, first):
    F = 1250
    body = functools.partial(_tc9_mid_body, first=first)
    return pl.pallas_call(
        body,
        out_shape=[SDS((F, 128), f32), SDS((F, 128), f32),
                   SDS((2, F, 128), f32)],
    )(y0f, y2f, p1p, qp, h1f, m0, mm, mp, cb2, bd0, bd12)


def _tc9_last_body(y0_ref, y2_ref, p1_ref, q_ref,
                   m0_ref, mm_ref, mp_ref, cb_ref,
                   bdl0_ref, lb0_ref, lgh_ref, lbt_ref, bdl1_ref, lb1_ref,
                   out_ref):
    cheb = (y0_ref[...] + p1_ref[0] + p1_ref[1]
            + 2.0 * (q_ref[0] + q_ref[1]) - y2_ref[...])
    a = jnp.maximum(cheb, 0.0)
    jk = _conv_flat(a, m0_ref[...], mm_ref[...], mp_ref[...]) + cb_ref[...]
    x = jnp.dot(jk, bdl0_ref[...], preferred_element_type=f32) + lb0_ref[...]
    x = jnp.maximum(x, 0.0)
    x = x * lgh_ref[...] + lbt_ref[...]
    out_ref[...] = (jnp.dot(x, bdl1_ref[...], preferred_element_type=f32)
                    + lb1_ref[...])


def _tc9_last(y0f, y2f, p1p, qp, m0, mm, mp, cb2,
              bdl0, lb0t, lght, lbtt, bdl1, lb1t):
    F = 1250
    return pl.pallas_call(
        _tc9_last_body,
        out_shape=SDS((F, 16), f32),
    )(y0f, y2f, p1p, qp, m0, mm, mp, cb2, bdl0, lb0t, lght, lbtt, bdl1, lb1t)


_prop_pass1 = functools.cache(lambda: _make_prop(combine=False))
_prop_pass2 = functools.cache(lambda: _make_prop(combine=True))


def _blockdiag(W, reps):
    bi, bo = W.shape
    out = jnp.zeros((reps * bi, reps * bo), f32)
    for j in range(reps):
        out = out.at[j * bi:(j + 1) * bi, j * bo:(j + 1) * bo].set(W)
    return out


def _conv_mats(cw, cb):
    A, B_, Cm = cw[:, :, 0].T, cw[:, :, 1].T, cw[:, :, 2].T
    m0 = jnp.zeros((128, 128), f32)
    mm = jnp.zeros((128, 128), f32)
    mp = jnp.zeros((128, 128), f32)
    for j in range(8):
        m0 = m0.at[j * 16:(j + 1) * 16, j * 16:(j + 1) * 16].set(B_)
        if j >= 1:
            m0 = m0.at[(j - 1) * 16:j * 16, j * 16:(j + 1) * 16].set(A)
        if j <= 6:
            m0 = m0.at[(j + 1) * 16:(j + 2) * 16, j * 16:(j + 1) * 16].set(Cm)
    mm = mm.at[7 * 16:, 0:16].set(A)
    mp = mp.at[0:16, 7 * 16:].set(Cm)
    cb2 = 2.0 * jnp.tile(cb, 8)[None]
    return 2.0 * m0, 2.0 * mm, 2.0 * mp, cb2


def kernel(image_features, edge_index, non_image_features,
           edge_index1, edge_weight1, params):
    p = params
    src = jnp.pad(edge_index[0].astype(i32), (0, E_PAD - E))
    dst = jnp.pad(edge_index[1].astype(i32), (0, E_PAD - E))
    nim = jnp.pad(non_image_features, ((0, E_PAD - E), (0, 0)))

    wcat0 = jnp.concatenate([p['Wch0'][0], p['Wch0'][1], p['Wch0'][2]], axis=1)

    ei, nsq2, y0_0, t1_0 = _tc1(image_features, p['Wei0'], p['bei0'],
                                p['Wei1'], p['bei1'], wcat0)
    nsq = nsq2.reshape(N)

    dote, nsqs, nsqd = _sc2_kernel()(ei, nsq, src, dst)

    ew2 = _tc3(nim, dote.reshape(E_PAD, 1), nsqs.reshape(E_PAD, 1),
               nsqd.reshape(E_PAD, 1), p)
    ew = ew2.reshape(E_PAD)

    degp = _sc4_kernel()(src, ew)
    dis = _tc5(degp).reshape(N)
    norm = _sc6_kernel()(dis, src, dst, ew)
    dst3 = dst.reshape(NW, NCH, C)

    y0f = y0_0.reshape(1250, 128)
    t1_tab = t1_0.reshape(2 * N, 16)
    y2f = t1_0.reshape(2, 1250, 128)[1]

    h1f = None
    out = None
    for i in range(4):
        p1p, zp = _prop_pass1()(t1_tab, src, dst3, norm)
        qp = _prop_pass2()(zp.reshape(NC * N, 16), src, dst3, norm)
        p1pf = p1p.reshape(2, 1250, 128)
        qpf = qp.reshape(2, 1250, 128)
        cwm = _conv_mats(p['cw%d' % i], p['cb%d' % i])
        if i < 3:
            wn = p['Wch%d' % (i + 1)]
            bd0 = _blockdiag(wn[0], 8)
            bd12 = jnp.concatenate(
                [_blockdiag(wn[1], 8), _blockdiag(wn[2], 8)], axis=1)
            h1_in = h1f if h1f is not None else y0f
            h1f, y0f, t1v = _tc9_mid(y0f, y2f, p1pf, qpf, h1_in,
                                     *cwm, bd0, bd12, first=(i == 0))
            t1_tab = t1v.reshape(2 * N, 16)
            y2f = t1v[1]
        else:
            lgh = p['lg'] / jnp.sqrt(1.0 + _BN_EPS)
            out = _tc9_last(y0f, y2f, p1pf, qpf, *cwm,
                            _blockdiag(p['lW0'], 8),
                            jnp.tile(p['lb0'], 8)[None],
                            jnp.tile(lgh, 8)[None],
                            jnp.tile(p['lbt'], 8)[None],
                            _blockdiag(p['lW1'], 8),
                            jnp.tile(p['lb1'], 8)[None])
    return out.reshape(N, 2)


# speedup vs baseline: 6.1462x; 1.0373x over previous
import functools

import jax
import jax.numpy as jnp
import numpy as np
from jax import lax
from jax.experimental import pallas as pl
from jax.experimental.pallas import tpu as pltpu
from jax.experimental.pallas import tpu_sc as plsc

N = 10000
E = 160000
IN_DIM = 512
EMB = 256
HID = 16

NC, NS, L = 2, 16, 16
NW = NC * NS
E_PAD = 163840
EPW = E_PAD // NW
C = 128
NCH = EPW // C
NPT = N // NS

_BN_EPS = 1e-5

_CP = pltpu.CompilerParams(use_tc_tiling_on_sc=False, needs_layout_passes=False)


@functools.cache
def _mesh():
    return plsc.VectorSubcoreMesh(core_axis_name="c", subcore_axis_name="s")

f32 = jnp.float32
i32 = jnp.int32
SDS = jax.ShapeDtypeStruct


def _wid():
    return lax.axis_index("s") * NC + lax.axis_index("c")


def _tc1_body(x_ref, w0_ref, b0_ref, w1_ref, b1_ref, wc_ref,
              ei_ref, nsq_ref, y0_ref, t1_ref):
    x = x_ref[...]
    hp = jax.lax.Precision.HIGHEST
    h = jnp.dot(x, w0_ref[...], preferred_element_type=f32,
                precision=hp) + b0_ref[...]
    ei = jnp.dot(h, w1_ref[...], preferred_element_type=f32,
                 precision=hp) + b1_ref[...]
    ei_ref[...] = ei
    nsq_ref[...] = jnp.sum(ei * ei, axis=1, keepdims=True)
    wc = wc_ref[...]
    y = jnp.dot(ei, wc, preferred_element_type=f32)
    y0_ref[...] = y[:, 0:16]
    t1_ref[0] = y[:, 16:32]
    t1_ref[1] = y[:, 32:48]


def _tc1(x, w0, b0, w1, b1, wcat):
    R = 1000
    return pl.pallas_call(
        _tc1_body,
        grid=(N // R,),
        in_specs=[
            pl.BlockSpec((R, IN_DIM), lambda i: (i, 0)),
            pl.BlockSpec((IN_DIM, IN_DIM), lambda i: (0, 0)),
            pl.BlockSpec((1, IN_DIM), lambda i: (0, 0)),
            pl.BlockSpec((IN_DIM, EMB), lambda i: (0, 0)),
            pl.BlockSpec((1, EMB), lambda i: (0, 0)),
            pl.BlockSpec((EMB, 48), lambda i: (0, 0)),
        ],
        out_specs=[
            pl.BlockSpec((R, EMB), lambda i: (i, 0)),
            pl.BlockSpec((R, 1), lambda i: (i, 0)),
            pl.BlockSpec((R, 16), lambda i: (i, 0)),
            pl.BlockSpec((2, R, 16), lambda i: (0, i, 0)),
        ],
        out_shape=[
            SDS((N, EMB), f32),
            SDS((N, 1), f32),
            SDS((N, 16), f32),
            SDS((2, N, 16), f32),
        ],
    )(x, w0, b0[None], w1, b1[None], wcat)


@functools.cache
def _sc2_kernel():
  deco = functools.partial(
    pl.kernel, mesh=_mesh(), compiler_params=_CP,
    out_type=(SDS((E_PAD,), f32), SDS((E_PAD,), f32), SDS((E_PAD,), f32)),
    scratch_types=[
        pltpu.VMEM((N,), f32),
        pltpu.VMEM((EPW,), i32),
        pltpu.VMEM((EPW,), i32),
        pltpu.VMEM((EPW,), f32),
        pltpu.VMEM((EPW,), f32),
        pltpu.VMEM((EPW,), f32),
        pltpu.VMEM((2, C, EMB), jnp.bfloat16),
        pltpu.VMEM((2, C, EMB), jnp.bfloat16),
        pltpu.VMEM((C, 16), f32),
        pltpu.SemaphoreType.DMA,
        pltpu.SemaphoreType.DMA,
        pltpu.SemaphoreType.DMA,
        pltpu.SemaphoreType.DMA,
    ],
)
  def _sc2(es_hbm, nsq_hbm, src_hbm, dst_hbm,
           dot_o, nsqs_o, nsqd_o,
           nsqb, srcb, dstb, dots, nsqsb, nsqdb, rs, rd, dtmp,
           gs0, gs1, gd0, gd1):
      base = _wid() * EPW
      pltpu.sync_copy(nsq_hbm, nsqb)
      pltpu.sync_copy(src_hbm.at[pl.ds(base, EPW)], srcb)
      pltpu.sync_copy(dst_hbm.at[pl.ds(base, EPW)], dstb)

      @pl.loop(0, EPW // 16)
      def _(g):
          s16 = srcb[pl.ds(g * 16, 16)]
          d16 = dstb[pl.ds(g * 16, 16)]
          nsqsb[pl.ds(g * 16, 16)] = plsc.load_gather(nsqb, [s16])
          nsqdb[pl.ds(g * 16, 16)] = plsc.load_gather(nsqb, [d16])

      gsems = (gs0, gs1)
      gdems = (gd0, gd1)
      for b in range(2):
          pltpu.async_copy(es_hbm.at[srcb.at[pl.ds(b * C, C)]], rs.at[b], gsems[b])
          pltpu.async_copy(es_hbm.at[dstb.at[pl.ds(b * C, C)]], rd.at[b], gdems[b])

      @pl.loop(0, NCH // 2)
      def _(k2):
          for b in range(2):
              k = k2 * 2 + b
              pltpu.make_async_copy(es_hbm.at[srcb.at[pl.ds(k * C, C)]],
                                    rs.at[b], gsems[b]).wait()
              pltpu.make_async_copy(es_hbm.at[dstb.at[pl.ds(k * C, C)]],
                                    rd.at[b], gdems[b]).wait()
              rs_b = rs.at[b]
              rd_b = rd.at[b]

              @pl.loop(0, C)
              def _(e):
                  acc = jnp.zeros((16,), f32)
                  for q in range(EMB // 32):
                      sa = rs_b[e, pl.ds(q * 32, 32)]
                      da = rd_b[e, pl.ds(q * 32, 32)]
                      s1, s2 = plsc.unpack(
                          sa, format=plsc.PackFormat.INTERLEAVED,
                          preferred_element_type=f32)
                      d1, d2 = plsc.unpack(
                          da, format=plsc.PackFormat.INTERLEAVED,
                          preferred_element_type=f32)
                      acc = acc + s1 * d1 + s2 * d2
                  dtmp[e] = plsc.cumsum(acc)

              lane15 = jnp.full((16,), 15, i32)
              ramp = lax.iota(i32, 16)
              for g in range(C // 16):
                  dots[pl.ds(k * C + g * 16, 16)] = plsc.load_gather(
                      dtmp, [ramp + g * 16, lane15])

              @pl.when(k + 2 < NCH)
              def _():
                  pltpu.async_copy(es_hbm.at[srcb.at[pl.ds((k + 2) * C, C)]],
                                   rs.at[b], gsems[b])
                  pltpu.async_copy(es_hbm.at[dstb.at[pl.ds((k + 2) * C, C)]],
                                   rd.at[b], gdems[b])

      pltpu.sync_copy(dots, dot_o.at[pl.ds(base, EPW)])
      pltpu.sync_copy(nsqsb, nsqs_o.at[pl.ds(base, EPW)])
      pltpu.sync_copy(nsqdb, nsqd_o.at[pl.ds(base, EPW)])
  return deco(_sc2)


def _tc3_body(nim_ref, dot_ref, nsqs_ref, nsqd_ref,
              w0_ref, b0_ref, gh_ref, bt_ref, w1_ref, b1_ref,
              ew_ref):
    i = pl.program_id(0)
    x = nim_ref[...]
    w0 = w0_ref[...]
    b0 = b0_ref[...]
    gh = gh_ref[...]
    bt = bt_ref[...]
    w1 = w1_ref[...]
    b1 = b1_ref[...]

    def parser(a, b):
        h = jnp.maximum(a * w0[0:1, :] + b * w0[1:2, :] + b0, 0.0)
        h = h * gh + bt
        return jnp.dot(h, w1, preferred_element_type=f32) + b1

    p1 = parser(x[:, 0:1], x[:, 1:2])
    p2 = parser(x[:, 2:3], x[:, 3:4])
    s12 = jnp.sum(p1 * p2, axis=1, keepdims=True)
    s11 = jnp.sum(p1 * p1, axis=1, keepdims=True)
    s22 = jnp.sum(p2 * p2, axis=1, keepdims=True)
    num = s12 + dot_ref[...]
    n1 = jnp.maximum(jnp.sqrt(s11 + nsqs_ref[...]), 1e-8)
    n2 = jnp.maximum(jnp.sqrt(s22 + nsqd_ref[...]), 1e-8)
    ew = (num / (n1 * n2) + 1.0) * 0.5
    B = ew.shape[0]
    ridx = i * B + lax.broadcasted_iota(i32, (B, 1), 0)
    ew_ref[...] = jnp.where(ridx < E, ew, 0.0)


def _tc3(nim, dote, nsqs, nsqd, p):
    B = 2048
    gh = (p['pg'] / jnp.sqrt(1.0 + _BN_EPS))[None]
    col = lambda i: (i, 0)
    return pl.pallas_call(
        _tc3_body,
        grid=(E_PAD // B,),
        in_specs=[
            pl.BlockSpec((B, 4), col),
            pl.BlockSpec((B, 1), col),
            pl.BlockSpec((B, 1), col),
            pl.BlockSpec((B, 1), col),
            pl.BlockSpec((2, 128), lambda i: (0, 0)),
            pl.BlockSpec((1, 128), lambda i: (0, 0)),
            pl.BlockSpec((1, 128), lambda i: (0, 0)),
            pl.BlockSpec((1, 128), lambda i: (0, 0)),
            pl.BlockSpec((128, 128), lambda i: (0, 0)),
            pl.BlockSpec((1, 128), lambda i: (0, 0)),
        ],
        out_specs=pl.BlockSpec((B, 1), col),
        out_shape=SDS((E_PAD, 1), f32),
    )(nim, dote, nsqs, nsqd, p['pW0'], p['pb0'][None], gh, p['pbt'][None],
      p['pW1'], p['pb1'][None])


@functools.cache
def _sc4_kernel():
  deco = functools.partial(
    pl.kernel, mesh=_mesh(), compiler_params=_CP,
    out_type=SDS((NW, N), f32),
    scratch_types=[
        pltpu.VMEM((N,), f32),
        pltpu.VMEM((EPW,), i32),
        pltpu.VMEM((EPW,), f32),
    ],
)
  def _sc4(src_hbm, ew_hbm, degp_o, degb, srcb, ewb):
      w = _wid()
      base = w * EPW
      pltpu.sync_copy(src_hbm.at[pl.ds(base, EPW)], srcb)
      pltpu.sync_copy(ew_hbm.at[pl.ds(base, EPW)], ewb)

      @pl.loop(0, N // 16)
      def _(r):
          degb[pl.ds(r * 16, 16)] = jnp.zeros((16,), f32)

      @pl.loop(0, EPW // 16)
      def _(g):
          idx = srcb[pl.ds(g * 16, 16)]
          val = ewb[pl.ds(g * 16, 16)]
          plsc.addupdate_scatter(degb, [idx], val)

      pltpu.sync_copy(degb, degp_o.at[w])
  return deco(_sc4)


def _tc5_body(degp_ref, dis_ref):
    deg = jnp.sum(degp_ref[...], axis=0, keepdims=True)
    safe = jnp.where(deg > 0, deg, 1.0)
    dis_ref[...] = jnp.where(deg > 0, lax.rsqrt(safe), 0.0)


def _tc5(degp):
    return pl.pallas_call(
        _tc5_body,
        out_shape=SDS((1, N), f32),
    )(degp)


@functools.cache
def _sc6_kernel():
  deco = functools.partial(
    pl.kernel, mesh=_mesh(), compiler_params=_CP,
    out_type=SDS((E_PAD,), f32),
    scratch_types=[
        pltpu.VMEM((N,), f32),
        pltpu.VMEM((EPW,), i32),
        pltpu.VMEM((EPW,), i32),
        pltpu.VMEM((EPW,), f32),
        pltpu.VMEM((EPW,), f32),
    ],
)
  def _sc6(dis_hbm, src_hbm, dst_hbm, ew_hbm, norm_o,
           disb, srcb, dstb, ewb, normb):
      base = _wid() * EPW
      pltpu.sync_copy(dis_hbm, disb)
      pltpu.sync_copy(src_hbm.at[pl.ds(base, EPW)], srcb)
      pltpu.sync_copy(dst_hbm.at[pl.ds(base, EPW)], dstb)
      pltpu.sync_copy(ew_hbm.at[pl.ds(base, EPW)], ewb)

      @pl.loop(0, EPW // 16)
      def _(g):
          s16 = srcb[pl.ds(g * 16, 16)]
          d16 = dstb[pl.ds(g * 16, 16)]
          a = plsc.load_gather(disb, [s16])
          b = plsc.load_gather(disb, [d16])
          normb[pl.ds(g * 16, 16)] = -a * ewb[pl.ds(g * 16, 16)] * b

      pltpu.sync_copy(normb, norm_o.at[pl.ds(base, EPW)])
  return deco(_sc6)


def _make_prop(combine):
    n_acc = 1 if combine else 2
    out_t = (SDS((NC, N, 16), f32) if combine
             else (SDS((NC, N, 16), f32), SDS((NC, N, 16), f32)))
    scr = [
        pltpu.VMEM((EPW,), i32),
        pltpu.VMEM((EPW,), i32),
        pltpu.VMEM((NCH, C), i32),
        pltpu.VMEM((EPW,), f32),
        pltpu.VMEM((NPT, 16), f32),
        pltpu.VMEM((2, C, 16), f32),
        pltpu.VMEM((2, C, 16), f32),
        pltpu.VMEM((2, C, 16), f32),
    ]
    if not combine:
        scr.append(pltpu.VMEM((2, C, 16), f32))
    scr += [pltpu.SemaphoreType.DMA] * (6 if combine else 8)
    scr.append(pltpu.VMEM_SHARED((N, 16), f32))
    if not combine:
        scr.append(pltpu.VMEM_SHARED((N, 16), f32))

    def body(table_hbm, src_hbm, dst_hbm, norm_hbm, *rest):
        if combine:
            (out1, srcb, srcb2, dstb, normb, zbuf, ga, gb, s1,
             sga0, sga1, sgb0, sgb1, ss0, ss1, acc1) = rest
            outs, accs, sbufs = (out1,), (acc1,), (s1,)
            ssems = ((ss0, ss1),)
        else:
            (out1, out2, srcb, srcb2, dstb, normb, zbuf, ga, gb, s1, s2,
             sga0, sga1, sgb0, sgb1, ss0, ss1, ss20, ss21,
             acc1, acc2) = rest
            outs, accs, sbufs = (out1, out2), (acc1, acc2), (s1, s2)
            ssems = ((ss0, ss1), (ss20, ss21))

        w = _wid()
        cid = lax.axis_index("c")
        sid = lax.axis_index("s")
        base = w * EPW

        pltpu.sync_copy(src_hbm.at[pl.ds(base, EPW)], srcb)
        pltpu.sync_copy(norm_hbm.at[pl.ds(base, EPW)], normb)
        pltpu.sync_copy(dst_hbm.at[w], dstb)

        @pl.loop(0, EPW // 16)
        def _(g):
            srcb2[pl.ds(g * 16, 16)] = srcb[pl.ds(g * 16, 16)] + N

        @pl.loop(0, NPT)
        def _(r):
            zbuf[r] = jnp.zeros((16,), f32)

        for acc in accs:
            pltpu.sync_copy(zbuf, acc.at[pl.ds(sid * NPT, NPT)])
        plsc.subcore_barrier()

        gsa = (sga0, sga1)
        gsb = (sgb0, sgb1)

        for b in range(2):
            pltpu.async_copy(table_hbm.at[srcb.at[pl.ds(b * C, C)]],
                             ga.at[b], gsa[b])
            pltpu.async_copy(table_hbm.at[srcb2.at[pl.ds(b * C, C)]],
                             gb.at[b], gsb[b])

        @pl.loop(0, NCH // 2)
        def _(k2):
            for b in range(2):
                k = k2 * 2 + b
                pltpu.make_async_copy(table_hbm.at[srcb.at[pl.ds(k * C, C)]],
                                      ga.at[b], gsa[b]).wait()
                pltpu.make_async_copy(table_hbm.at[srcb2.at[pl.ds(k * C, C)]],
                                      gb.at[b], gsb[b]).wait()

                @pl.when(k >= 2)
                def _():
                    for ai in range(n_acc):
                        pltpu.make_async_copy(
                            sbufs[ai].at[b],
                            accs[ai].at[dstb.at[k - 2]], ssems[ai][b]).wait()

                ga_b = ga.at[b]
                gb_b = gb.at[b]
                s1_b = sbufs[0].at[b]
                if not combine:
                    s2_b = sbufs[1].at[b]

                @pl.loop(0, C, unroll=8)
                def _(e):
                    nb = plsc.load_gather(
                        normb, [jnp.full((16,), 1, i32) * (k * C + e)])
                    if combine:
                        s1_b[e] = (ga_b[e] + gb_b[e]) * nb
                    else:
                        s1_b[e] = ga_b[e] * nb
                        s2_b[e] = gb_b[e] * nb

                @pl.when(k + 2 < NCH)
                def _():
                    pltpu.async_copy(
                        table_hbm.at[srcb.at[pl.ds((k + 2) * C, C)]],
                        ga.at[b], gsa[b])
                    pltpu.async_copy(
                        table_hbm.at[srcb2.at[pl.ds((k + 2) * C, C)]],
                        gb.at[b], gsb[b])

                for ai in range(n_acc):
                    pltpu.async_copy(sbufs[ai].at[b],
                                     accs[ai].at[dstb.at[k]], ssems[ai][b],
                                     add=True)

        for b in range(2):
            k = NCH - 2 + b
            for ai in range(n_acc):
                pltpu.make_async_copy(sbufs[ai].at[b],
                                      accs[ai].at[dstb.at[k]],
                                      ssems[ai][b]).wait()
        plsc.subcore_barrier()
        for ai in range(n_acc):
            pltpu.sync_copy(accs[ai].at[pl.ds(sid * NPT, NPT)],
                            outs[ai].at[cid].at[pl.ds(sid * NPT, NPT)])

    return functools.partial(pl.kernel, mesh=_mesh(), compiler_params=_CP,
                             out_type=out_t, scratch_types=scr)(body)


def _conv_flat(a, m0, mm, mp):
    up = jnp.concatenate([jnp.zeros((1, 128), f32), a[:-1]], axis=0)
    dn = jnp.concatenate([a[1:], jnp.zeros((1, 128), f32)], axis=0)
    return (jnp.dot(a, m0, preferred_element_type=f32)
            + jnp.dot(up, mm, preferred_element_type=f32)
            + jnp.dot(dn, mp, preferred_element_type=f32))


def _tc9_mid_body(y0_ref, y2_ref, p1_ref, q_ref, h1_ref,
                  m0_ref, mm_ref, mp_ref, cb_ref, bd0_ref, bd12_ref,
                  h1o_ref, y0o_ref, t1o_ref, first):
    cheb = (y0_ref[...] + p1_ref[0] + p1_ref[1]
            + 2.0 * (q_ref[0] + q_ref[1]) - y2_ref[...])
    a = jnp.maximum(cheb, 0.0)
    hh = _conv_flat(a, m0_ref[...], mm_ref[...], mp_ref[...]) + cb_ref[...]
    h1 = hh if first else h1_ref[...] + hh
    h1o_ref[...] = h1
    y0o_ref[...] = jnp.dot(h1, bd0_ref[...], preferred_element_type=f32)
    t = jnp.dot(h1, bd12_ref[...], preferred_element_type=f32)
    t1o_ref[0] = t[:, 0:128]
    t1o_ref[1] = t[:, 128:256]


def _tc9_mid(y0f, y2f, p1p, qp, h1f, m0, mm, mp, cb2, bd0, bd12, first):
    F = 1250
    body = functools.partial(_tc9_mid_body, first=first)
    return pl.pallas_call(
        body,
        out_shape=[SDS((F, 128), f32), SDS((F, 128), f32),
                   SDS((2, F, 128), f32)],
    )(y0f, y2f, p1p, qp, h1f, m0, mm, mp, cb2, bd0, bd12)


def _tc9_last_body(y0_ref, y2_ref, p1_ref, q_ref,
                   m0_ref, mm_ref, mp_ref, cb_ref,
                   bdl0_ref, lb0_ref, lgh_ref, lbt_ref, bdl1_ref, lb1_ref,
                   out_ref):
    cheb = (y0_ref[...] + p1_ref[0] + p1_ref[1]
            + 2.0 * (q_ref[0] + q_ref[1]) - y2_ref[...])
    a = jnp.maximum(cheb, 0.0)
    jk = _conv_flat(a, m0_ref[...], mm_ref[...], mp_ref[...]) + cb_ref[...]
    x = jnp.dot(jk, bdl0_ref[...], preferred_element_type=f32) + lb0_ref[...]
    x = jnp.maximum(x, 0.0)
    x = x * lgh_ref[...] + lbt_ref[...]
    out_ref[...] = (jnp.dot(x, bdl1_ref[...], preferred_element_type=f32)
                    + lb1_ref[...])


def _tc9_last(y0f, y2f, p1p, qp, m0, mm, mp, cb2,
              bdl0, lb0t, lght, lbtt, bdl1, lb1t):
    F = 1250
    return pl.pallas_call(
        _tc9_last_body,
        out_shape=SDS((F, 16), f32),
    )(y0f, y2f, p1p, qp, m0, mm, mp, cb2, bdl0, lb0t, lght, lbtt, bdl1, lb1t)


_prop_pass1 = functools.cache(lambda: _make_prop(combine=False))
_prop_pass2 = functools.cache(lambda: _make_prop(combine=True))


def _blockdiag(W, reps):
    bi, bo = W.shape
    out = jnp.zeros((reps * bi, reps * bo), f32)
    for j in range(reps):
        out = out.at[j * bi:(j + 1) * bi, j * bo:(j + 1) * bo].set(W)
    return out


def _conv_mats(cw, cb):
    A, B_, Cm = cw[:, :, 0].T, cw[:, :, 1].T, cw[:, :, 2].T
    m0 = jnp.zeros((128, 128), f32)
    mm = jnp.zeros((128, 128), f32)
    mp = jnp.zeros((128, 128), f32)
    for j in range(8):
        m0 = m0.at[j * 16:(j + 1) * 16, j * 16:(j + 1) * 16].set(B_)
        if j >= 1:
            m0 = m0.at[(j - 1) * 16:j * 16, j * 16:(j + 1) * 16].set(A)
        if j <= 6:
            m0 = m0.at[(j + 1) * 16:(j + 2) * 16, j * 16:(j + 1) * 16].set(Cm)
    mm = mm.at[7 * 16:, 0:16].set(A)
    mp = mp.at[0:16, 7 * 16:].set(Cm)
    cb2 = 2.0 * jnp.tile(cb, 8)[None]
    return 2.0 * m0, 2.0 * mm, 2.0 * mp, cb2


def kernel(image_features, edge_index, non_image_features,
           edge_index1, edge_weight1, params):
    p = params
    src = jnp.pad(edge_index[0].astype(i32), (0, E_PAD - E))
    dst = jnp.pad(edge_index[1].astype(i32), (0, E_PAD - E))
    nim = jnp.pad(non_image_features, ((0, E_PAD - E), (0, 0)))

    wcat0 = jnp.concatenate([p['Wch0'][0], p['Wch0'][1], p['Wch0'][2]], axis=1)

    ei, nsq2, y0_0, t1_0 = _tc1(image_features, p['Wei0'], p['bei0'],
                                p['Wei1'], p['bei1'], wcat0)
    nsq = nsq2.reshape(N)
    es16 = ei.astype(jnp.bfloat16)

    dote, nsqs, nsqd = _sc2_kernel()(es16, nsq, src, dst)

    ew2 = _tc3(nim, dote.reshape(E_PAD, 1), nsqs.reshape(E_PAD, 1),
               nsqd.reshape(E_PAD, 1), p)
    ew = ew2.reshape(E_PAD)

    degp = _sc4_kernel()(src, ew)
    dis = _tc5(degp).reshape(N)
    norm = _sc6_kernel()(dis, src, dst, ew)
    dst3 = dst.reshape(NW, NCH, C)

    y0f = y0_0.reshape(1250, 128)
    t1_tab = t1_0.reshape(2 * N, 16)
    y2f = t1_0.reshape(2, 1250, 128)[1]

    h1f = None
    out = None
    for i in range(4):
        p1p, zp = _prop_pass1()(t1_tab, src, dst3, norm)
        qp = _prop_pass2()(zp.reshape(NC * N, 16), src, dst3, norm)
        p1pf = p1p.reshape(2, 1250, 128)
        qpf = qp.reshape(2, 1250, 128)
        cwm = _conv_mats(p['cw%d' % i], p['cb%d' % i])
        if i < 3:
            wn = p['Wch%d' % (i + 1)]
            bd0 = _blockdiag(wn[0], 8)
            bd12 = jnp.concatenate(
                [_blockdiag(wn[1], 8), _blockdiag(wn[2], 8)], axis=1)
            h1_in = h1f if h1f is not None else y0f
            h1f, y0f, t1v = _tc9_mid(y0f, y2f, p1pf, qpf, h1_in,
                                     *cwm, bd0, bd12, first=(i == 0))
            t1_tab = t1v.reshape(2 * N, 16)
            y2f = t1v[1]
        else:
            lgh = p['lg'] / jnp.sqrt(1.0 + _BN_EPS)
            out = _tc9_last(y0f, y2f, p1pf, qpf, *cwm,
                            _blockdiag(p['lW0'], 8),
                            jnp.tile(p['lb0'], 8)[None],
                            jnp.tile(lgh, 8)[None],
                            jnp.tile(p['lbt'], 8)[None],
                            _blockdiag(p['lW1'], 8),
                            jnp.tile(p['lb1'], 8)[None])
    return out.reshape(N, 2)
